# baseline (speedup 1.0000x reference)
import jax
import jax.numpy as jnp
from jax import lax
from jax.experimental import pallas as pl
from jax.experimental.pallas import tpu as pltpu
from jax.experimental.pallas import tpu_sc as plsc

ROWS = 128
N = 100000
NALIGNED = (N // 128) * 128
NTAIL = N - NALIGNED
BR = 16
NBLK = ROWS // BR

_BIG_I32 = 2**30


def _stats_kernel(x_ref, u_ref, ent_ref, idx_ref):
    xb = x_ref[...]
    ub = u_ref[...]

    col = lax.broadcasted_iota(jnp.int32, (BR, N), 1)

    uc = jnp.clip(ub, 1e-10, 1.0 - 1e-10)
    y = xb - jnp.log(-jnp.log(uc))

    lv = jnp.max(y, axis=1, keepdims=True)
    li = jnp.min(jnp.where(y == lv, col, _BIG_I32), axis=1, keepdims=True)
    idx_ref[...] = li

    m = jnp.max(xb, axis=1, keepdims=True)
    e = jnp.exp(xb - m)
    z = jnp.sum(e, axis=1, keepdims=True)
    w = jnp.sum(xb * e, axis=1, keepdims=True)
    ent_ref[...] = m + jnp.log(z) - w / z


def _stats_pass(x, gumbel_u):
    return pl.pallas_call(
        _stats_kernel,
        grid=(NBLK,),
        in_specs=[
            pl.BlockSpec((BR, N), lambda i: (i, 0)),
            pl.BlockSpec((BR, N), lambda i: (i, 0)),
        ],
        out_specs=[
            pl.BlockSpec((BR, 1), lambda i: (i, 0)),
            pl.BlockSpec((BR, 1), lambda i: (i, 0)),
        ],
        out_shape=[
            jax.ShapeDtypeStruct((ROWS, 1), jnp.float32),
            jax.ShapeDtypeStruct((ROWS, 1), jnp.int32),
        ],
    )(x, gumbel_u)


CHUNK = 12800
_CHUNKS = [(c0, min(CHUNK, NALIGNED - c0)) for c0 in range(0, NALIGNED, CHUNK)]
_GROUPS = ROWS // 8


def _sc_fill_body(x_hbm, sample_hbm, scores_hbm, vbuf):
    cid = lax.axis_index("c")
    sid = lax.axis_index("s")
    wid = sid * 2 + cid
    row = jnp.where(wid < _GROUPS, wid, wid - _GROUPS) * 8

    @pl.when(wid < _GROUPS)
    def _copy():
        for c0, w in _CHUNKS:
            pltpu.sync_copy(x_hbm.at[pl.ds(row, 8), pl.ds(c0, w)],
                            vbuf.at[:, pl.ds(0, w)])
            pltpu.sync_copy(vbuf.at[:, pl.ds(0, w)],
                            scores_hbm.at[pl.ds(row, 8), pl.ds(c0, w)])

    @pl.when(wid >= _GROUPS)
    def _zero():
        zeros16 = jnp.zeros((16,), jnp.float32)
        for r in range(8):
            def zrow(j, carry):
                vbuf[r, pl.ds(j * 16, 16)] = zeros16
                return carry
            lax.fori_loop(0, CHUNK // 16, zrow, 0)
        for c0, w in _CHUNKS:
            pltpu.sync_copy(vbuf.at[:, pl.ds(0, w)],
                            sample_hbm.at[pl.ds(row, 8), pl.ds(c0, w)])


def _sc_fill(x):
    return pl.kernel(
        _sc_fill_body,
        out_type=[
            jax.ShapeDtypeStruct((ROWS, N), jnp.float32),
            jax.ShapeDtypeStruct((ROWS, N), jnp.float32),
        ],
        mesh=plsc.VectorSubcoreMesh(core_axis_name="c", subcore_axis_name="s"),
        scratch_types=[pltpu.VMEM((8, CHUNK), jnp.float32)],
    )(x)


def _scatter_kernel(idx_v_ref, x_tail_ref, idx_s_ref, zeroed_ref,
                    out_ref, sample_tail_ref, scores_tail_ref, obuf, sem):
    idx_v = idx_v_ref[...]
    lane = lax.broadcasted_iota(jnp.int32, (ROWS, 128), 1)
    in_main = idx_v < NALIGNED
    obuf[...] = jnp.where((lane == idx_v % 128) & in_main, 1.0, 0.0)

    tail_col = lax.broadcasted_iota(jnp.int32, (ROWS, NTAIL), 1) + NALIGNED
    sample_tail_ref[...] = jnp.where(tail_col == idx_v, 1.0, 0.0)
    scores_tail_ref[...] = x_tail_ref[...]

    def window(r):
        c0 = (idx_s_ref[r, 0] // 128) * 128
        return jnp.minimum(c0, NALIGNED - 128)

    def issue(r, carry):
        pltpu.make_async_copy(
            obuf.at[pl.ds(r, 1), :],
            out_ref.at[pl.ds(r, 1), pl.ds(window(r), 128)],
            sem,
        ).start()
        return carry

    lax.fori_loop(0, ROWS, issue, 0)

    def drain(r, carry):
        pltpu.make_async_copy(
            obuf.at[pl.ds(r, 1), :],
            out_ref.at[pl.ds(r, 1), pl.ds(window(r), 128)],
            sem,
        ).wait()
        return carry

    lax.fori_loop(0, ROWS, drain, 0)


def _scatter_ones(idx, x_tail, zeroed):
    return pl.pallas_call(
        _scatter_kernel,
        in_specs=[
            pl.BlockSpec((ROWS, 1), lambda: (0, 0)),
            pl.BlockSpec((ROWS, NTAIL), lambda: (0, 0)),
            pl.BlockSpec(memory_space=pltpu.SMEM),
            pl.BlockSpec(memory_space=pl.ANY),
        ],
        out_specs=[
            pl.BlockSpec(memory_space=pl.ANY),
            pl.BlockSpec((ROWS, NTAIL), lambda: (0, 0)),
            pl.BlockSpec((ROWS, NTAIL), lambda: (0, 0)),
        ],
        out_shape=[
            jax.ShapeDtypeStruct((ROWS, N), jnp.float32),
            jax.ShapeDtypeStruct((ROWS, NTAIL), jnp.float32),
            jax.ShapeDtypeStruct((ROWS, NTAIL), jnp.float32),
        ],
        input_output_aliases={3: 0},
        scratch_shapes=[
            pltpu.VMEM((ROWS, 128), jnp.float32),
            pltpu.SemaphoreType.DMA,
        ],
    )(idx, x_tail, idx, zeroed)


def kernel(x, gumbel_u):
    ent, idx = _stats_pass(x, gumbel_u)
    sample0, scores0 = _sc_fill(x)
    x_tail = lax.slice(x, (0, NALIGNED), (ROWS, N))
    sample1, sample_tail, scores_tail = _scatter_ones(idx, x_tail, sample0)
    sample = lax.dynamic_update_slice(sample1, sample_tail, (0, NALIGNED))
    scores = lax.dynamic_update_slice(scores0, scores_tail, (0, NALIGNED))
    return (sample, scores, ent.reshape(ROWS))


# speedup vs baseline: 1.0017x; 1.0017x over previous
import jax
import jax.numpy as jnp
from jax import lax
from jax.experimental import pallas as pl
from jax.experimental.pallas import tpu as pltpu
from jax.experimental.pallas import tpu_sc as plsc

ROWS = 128
N = 100000
NALIGNED = (N // 128) * 128
NTAIL = N - NALIGNED
BR = 16
NBLK = ROWS // BR

_BIG_I32 = 2**30


def _stats_kernel(x_ref, u_ref, ent_ref, idx_ref):
    xb = x_ref[...]
    ub = u_ref[...]

    col = lax.broadcasted_iota(jnp.int32, (BR, N), 1)

    uc = jnp.clip(ub, 1e-10, 1.0 - 1e-10)
    y = xb - jnp.log(-jnp.log(uc))

    lv = jnp.max(y, axis=1, keepdims=True)
    li = jnp.min(jnp.where(y == lv, col, _BIG_I32), axis=1, keepdims=True)
    idx_ref[...] = li

    m = jnp.max(xb, axis=1, keepdims=True)
    e = jnp.exp(xb - m)
    z = jnp.sum(e, axis=1, keepdims=True)
    w = jnp.sum(xb * e, axis=1, keepdims=True)
    ent_ref[...] = m + jnp.log(z) - w / z


def _stats_pass(x, gumbel_u):
    return pl.pallas_call(
        _stats_kernel,
        grid=(NBLK,),
        in_specs=[
            pl.BlockSpec((BR, N), lambda i: (i, 0)),
            pl.BlockSpec((BR, N), lambda i: (i, 0)),
        ],
        out_specs=[
            pl.BlockSpec((BR, 1), lambda i: (i, 0)),
            pl.BlockSpec((BR, 1), lambda i: (i, 0)),
        ],
        out_shape=[
            jax.ShapeDtypeStruct((ROWS, 1), jnp.float32),
            jax.ShapeDtypeStruct((ROWS, 1), jnp.int32),
        ],
    )(x, gumbel_u)


CHUNK = 12800
_CHUNKS = [(c0, min(CHUNK, NALIGNED - c0)) for c0 in range(0, NALIGNED, CHUNK)]
_GROUPS = ROWS // 8


def _sc_fill_body(x_hbm, sample_hbm, scores_hbm, vbuf):
    cid = lax.axis_index("c")
    sid = lax.axis_index("s")
    wid = sid * 2 + cid
    row = jnp.where(wid < _GROUPS, wid, wid - _GROUPS) * 8

    @pl.when(wid < _GROUPS)
    def _copy():
        for c0, w in _CHUNKS:
            pltpu.sync_copy(x_hbm.at[pl.ds(row, 8), pl.ds(c0, w)],
                            vbuf.at[:, pl.ds(0, w)])
            pltpu.sync_copy(vbuf.at[:, pl.ds(0, w)],
                            scores_hbm.at[pl.ds(row, 8), pl.ds(c0, w)])

    @pl.when(wid >= _GROUPS)
    def _zero():
        zeros16 = jnp.zeros((16,), jnp.float32)
        for r in range(8):
            def zrow(j, carry):
                vbuf[r, pl.ds(j * 16, 16)] = zeros16
                return carry
            lax.fori_loop(0, CHUNK // 16, zrow, 0)
        for c0, w in _CHUNKS:
            pltpu.sync_copy(vbuf.at[:, pl.ds(0, w)],
                            sample_hbm.at[pl.ds(row, 8), pl.ds(c0, w)])


def _sc_fill(x):
    return pl.kernel(
        _sc_fill_body,
        out_type=[
            jax.ShapeDtypeStruct((ROWS, N), jnp.float32),
            jax.ShapeDtypeStruct((ROWS, N), jnp.float32),
        ],
        mesh=plsc.VectorSubcoreMesh(core_axis_name="c", subcore_axis_name="s"),
        scratch_types=[pltpu.VMEM((8, CHUNK), jnp.float32)],
    )(x)


def _scatter_kernel(idx_v_ref, x_tail_ref, idx_s_ref, zeroed_ref,
                    out_ref, sample_tail_ref, scores_tail_ref, obuf, sem):
    idx_v = idx_v_ref[...]
    lane = lax.broadcasted_iota(jnp.int32, (ROWS, 128), 1)
    in_main = idx_v < NALIGNED
    obuf[...] = jnp.where((lane == idx_v % 128) & in_main, 1.0, 0.0)

    tail_col = lax.broadcasted_iota(jnp.int32, (ROWS, NTAIL), 1) + NALIGNED
    sample_tail_ref[...] = jnp.where(tail_col == idx_v, 1.0, 0.0)
    scores_tail_ref[...] = x_tail_ref[...]

    def window(r):
        c0 = (idx_s_ref[r, 0] // 128) * 128
        return jnp.minimum(c0, NALIGNED - 128)

    def issue(r, carry):
        pltpu.make_async_copy(
            obuf.at[pl.ds(r, 1), :],
            out_ref.at[pl.ds(r, 1), pl.ds(window(r), 128)],
            sem,
        ).start()
        return carry

    lax.fori_loop(0, ROWS, issue, 0)

    def drain(r, carry):
        pltpu.make_async_copy(
            obuf.at[pl.ds(r, 1), :],
            out_ref.at[pl.ds(r, 1), pl.ds(window(r), 128)],
            sem,
        ).wait()
        return carry

    lax.fori_loop(0, ROWS, drain, 0)


def _scatter_ones(idx, x_tail, zeroed):
    return pl.pallas_call(
        _scatter_kernel,
        in_specs=[
            pl.BlockSpec((ROWS, 1), lambda: (0, 0)),
            pl.BlockSpec((ROWS, NTAIL), lambda: (0, 0)),
            pl.BlockSpec(memory_space=pltpu.SMEM),
            pl.BlockSpec(memory_space=pl.ANY),
        ],
        out_specs=[
            pl.BlockSpec(memory_space=pl.ANY),
            pl.BlockSpec((ROWS, NTAIL), lambda: (0, 0)),
            pl.BlockSpec((ROWS, NTAIL), lambda: (0, 0)),
        ],
        out_shape=[
            jax.ShapeDtypeStruct((ROWS, N), jnp.float32),
            jax.ShapeDtypeStruct((ROWS, NTAIL), jnp.float32),
            jax.ShapeDtypeStruct((ROWS, NTAIL), jnp.float32),
        ],
        input_output_aliases={3: 0},
        scratch_shapes=[
            pltpu.VMEM((ROWS, 128), jnp.float32),
            pltpu.SemaphoreType.DMA,
        ],
    )(idx, x_tail, idx, zeroed)


def kernel(x, gumbel_u):
    sample0, scores0 = _sc_fill(x)
    ent, idx = _stats_pass(x, gumbel_u)
    x_tail = lax.slice(x, (0, NALIGNED), (ROWS, N))
    sample1, sample_tail, scores_tail = _scatter_ones(idx, x_tail, sample0)
    sample = lax.dynamic_update_slice(sample1, sample_tail, (0, NALIGNED))
    scores = lax.dynamic_update_slice(scores0, scores_tail, (0, NALIGNED))
    return (sample, scores, ent.reshape(ROWS))


# speedup vs baseline: 1.0117x; 1.0099x over previous
import jax
import jax.numpy as jnp
from jax import lax
from jax.experimental import pallas as pl
from jax.experimental.pallas import tpu as pltpu
from jax.experimental.pallas import tpu_sc as plsc

ROWS = 128
N = 100000
NALIGNED = (N // 128) * 128
NTAIL = N - NALIGNED
BR = 16
NBLK = ROWS // BR

_BIG_I32 = 2**30


def _stats_kernel(x_ref, u_ref, ent_ref, idx_ref):
    xb = x_ref[...]
    ub = u_ref[...]

    col = lax.broadcasted_iota(jnp.int32, (BR, N), 1)

    uc = jnp.clip(ub, 1e-10, 1.0 - 1e-10)
    y = xb - jnp.log(-jnp.log(uc))

    lv = jnp.max(y, axis=1, keepdims=True)
    li = jnp.min(jnp.where(y == lv, col, _BIG_I32), axis=1, keepdims=True)
    idx_ref[...] = li

    m = jnp.max(xb, axis=1, keepdims=True)
    e = jnp.exp(xb - m)
    z = jnp.sum(e, axis=1, keepdims=True)
    w = jnp.sum(xb * e, axis=1, keepdims=True)
    ent_ref[...] = m + jnp.log(z) - w / z


def _stats_pass(x, gumbel_u):
    return pl.pallas_call(
        _stats_kernel,
        grid=(NBLK,),
        in_specs=[
            pl.BlockSpec((BR, N), lambda i: (i, 0)),
            pl.BlockSpec((BR, N), lambda i: (i, 0)),
        ],
        out_specs=[
            pl.BlockSpec((BR, 1), lambda i: (i, 0)),
            pl.BlockSpec((BR, 1), lambda i: (i, 0)),
        ],
        out_shape=[
            jax.ShapeDtypeStruct((ROWS, 1), jnp.float32),
            jax.ShapeDtypeStruct((ROWS, 1), jnp.int32),
        ],
    )(x, gumbel_u)


CHUNK = 12800
_CHUNKS = [(c0, min(CHUNK, NALIGNED - c0)) for c0 in range(0, NALIGNED, CHUNK)]
_GROUPS = ROWS // 8


def _sc_fill_body(x_hbm, sample_hbm, scores_hbm, vbuf):
    cid = lax.axis_index("c")
    sid = lax.axis_index("s")
    wid = sid * 2 + cid
    row = jnp.where(wid < _GROUPS, wid, wid - _GROUPS) * 8

    @pl.when(wid < _GROUPS)
    def _copy():
        for c0, w in _CHUNKS:
            pltpu.sync_copy(x_hbm.at[pl.ds(row, 8), pl.ds(c0, w)],
                            vbuf.at[:, pl.ds(0, w)])
            pltpu.sync_copy(vbuf.at[:, pl.ds(0, w)],
                            scores_hbm.at[pl.ds(row, 8), pl.ds(c0, w)])

    @pl.when(wid >= _GROUPS)
    def _zero():
        zeros16 = jnp.zeros((16,), jnp.float32)
        for r in range(8):
            def zrow(j, carry):
                vbuf[r, pl.ds(j * 16, 16)] = zeros16
                return carry
            lax.fori_loop(0, CHUNK // 16, zrow, 0)
        for c0, w in _CHUNKS:
            pltpu.sync_copy(vbuf.at[:, pl.ds(0, w)],
                            sample_hbm.at[pl.ds(row, 8), pl.ds(c0, w)])


def _sc_fill(x):
    return pl.kernel(
        _sc_fill_body,
        out_type=[
            jax.ShapeDtypeStruct((ROWS, N), jnp.float32),
            jax.ShapeDtypeStruct((ROWS, N), jnp.float32),
        ],
        mesh=plsc.VectorSubcoreMesh(core_axis_name="c", subcore_axis_name="s"),
        scratch_types=[pltpu.VMEM((8, CHUNK), jnp.float32)],
    )(x)


def _scatter_kernel(idx_v_ref, idx_s_ref, x_any, zeroed_ref, scores0_ref,
                    sample_out, scores_out, obuf, stbuf, tbuf,
                    sem, sem_in, sem_tail):
    cp_in = pltpu.make_async_copy(
        x_any.at[:, pl.ds(NALIGNED, NTAIL)], tbuf, sem_in)
    cp_in.start()

    idx_v = idx_v_ref[...]
    lane = lax.broadcasted_iota(jnp.int32, (ROWS, 128), 1)
    in_main = idx_v < NALIGNED
    obuf[...] = jnp.where((lane == idx_v % 128) & in_main, 1.0, 0.0)

    tail_col = lax.broadcasted_iota(jnp.int32, (ROWS, NTAIL), 1) + NALIGNED
    stbuf[...] = jnp.where(tail_col == idx_v, 1.0, 0.0)
    cp_st = pltpu.make_async_copy(
        stbuf, sample_out.at[:, pl.ds(NALIGNED, NTAIL)], sem_tail)
    cp_st.start()

    def window(r):
        c0 = (idx_s_ref[r, 0] // 128) * 128
        return jnp.minimum(c0, NALIGNED - 128)

    def issue(r, carry):
        pltpu.make_async_copy(
            obuf.at[pl.ds(r, 1), :],
            sample_out.at[pl.ds(r, 1), pl.ds(window(r), 128)],
            sem,
        ).start()
        return carry

    lax.fori_loop(0, ROWS, issue, 0)

    cp_in.wait()
    cp_sc = pltpu.make_async_copy(
        tbuf, scores_out.at[:, pl.ds(NALIGNED, NTAIL)], sem_tail)
    cp_sc.start()

    def drain(r, carry):
        pltpu.make_async_copy(
            obuf.at[pl.ds(r, 1), :],
            sample_out.at[pl.ds(r, 1), pl.ds(window(r), 128)],
            sem,
        ).wait()
        return carry

    lax.fori_loop(0, ROWS, drain, 0)
    cp_st.wait()
    cp_sc.wait()


def _scatter_ones(idx, x, zeroed, scores0):
    return pl.pallas_call(
        _scatter_kernel,
        in_specs=[
            pl.BlockSpec((ROWS, 1), lambda: (0, 0)),
            pl.BlockSpec(memory_space=pltpu.SMEM),
            pl.BlockSpec(memory_space=pl.ANY),
            pl.BlockSpec(memory_space=pl.ANY),
            pl.BlockSpec(memory_space=pl.ANY),
        ],
        out_specs=[
            pl.BlockSpec(memory_space=pl.ANY),
            pl.BlockSpec(memory_space=pl.ANY),
        ],
        out_shape=[
            jax.ShapeDtypeStruct((ROWS, N), jnp.float32),
            jax.ShapeDtypeStruct((ROWS, N), jnp.float32),
        ],
        input_output_aliases={3: 0, 4: 1},
        scratch_shapes=[
            pltpu.VMEM((ROWS, 128), jnp.float32),
            pltpu.VMEM((ROWS, NTAIL), jnp.float32),
            pltpu.VMEM((ROWS, NTAIL), jnp.float32),
            pltpu.SemaphoreType.DMA,
            pltpu.SemaphoreType.DMA,
            pltpu.SemaphoreType.DMA,
        ],
    )(idx, idx, x, zeroed, scores0)


def kernel(x, gumbel_u):
    sample0, scores0 = _sc_fill(x)
    ent, idx = _stats_pass(x, gumbel_u)
    sample, scores = _scatter_ones(idx, x, sample0, scores0)
    return (sample, scores, ent.reshape(ROWS))


# speedup vs baseline: 1.0156x; 1.0039x over previous
import jax
import jax.numpy as jnp
from jax import lax
from jax.experimental import pallas as pl
from jax.experimental.pallas import tpu as pltpu
from jax.experimental.pallas import tpu_sc as plsc

ROWS = 128
N = 100000
NALIGNED = (N // 128) * 128
NTAIL = N - NALIGNED
BR = 16
NBLK = ROWS // BR

_BIG_I32 = 2**30


def _stats_kernel(x_ref, u_ref, ent_ref, idx_ref):
    xb = x_ref[...]
    ub = u_ref[...]

    col = lax.broadcasted_iota(jnp.int32, (BR, N), 1)

    uc = jnp.clip(ub, 1e-10, 1.0 - 1e-10)
    y = xb - jnp.log(-jnp.log(uc))

    lv = jnp.max(y, axis=1, keepdims=True)
    li = jnp.min(jnp.where(y == lv, col, _BIG_I32), axis=1, keepdims=True)
    idx_ref[...] = li

    m = jnp.max(xb, axis=1, keepdims=True)
    e = jnp.exp(xb - m)
    z = jnp.sum(e, axis=1, keepdims=True)
    w = jnp.sum(xb * e, axis=1, keepdims=True)
    ent_ref[...] = m + jnp.log(z) - w / z


def _stats_pass(x, gumbel_u):
    return pl.pallas_call(
        _stats_kernel,
        grid=(NBLK,),
        in_specs=[
            pl.BlockSpec((BR, N), lambda i: (i, 0)),
            pl.BlockSpec((BR, N), lambda i: (i, 0)),
        ],
        out_specs=[
            pl.BlockSpec((BR, 1), lambda i: (i, 0)),
            pl.BlockSpec((BR, 1), lambda i: (i, 0)),
        ],
        out_shape=[
            jax.ShapeDtypeStruct((ROWS, 1), jnp.float32),
            jax.ShapeDtypeStruct((ROWS, 1), jnp.int32),
        ],
    )(x, gumbel_u)


CHUNK = 12800
_CHUNKS = [(c0, min(CHUNK, NALIGNED - c0)) for c0 in range(0, NALIGNED, CHUNK)]
_GROUPS = ROWS // 8


def _sc_fill_body(x_hbm, sample_hbm, scores_hbm, vbuf):
    cid = lax.axis_index("c")
    sid = lax.axis_index("s")
    wid = sid * 2 + cid
    row = jnp.where(wid < _GROUPS, wid, wid - _GROUPS) * 8

    @pl.when(wid < _GROUPS)
    def _copy():
        for c0, w in _CHUNKS:
            pltpu.sync_copy(x_hbm.at[pl.ds(row, 8), pl.ds(c0, w)],
                            vbuf.at[:, pl.ds(0, w)])
            pltpu.sync_copy(vbuf.at[:, pl.ds(0, w)],
                            scores_hbm.at[pl.ds(row, 8), pl.ds(c0, w)])

    @pl.when(wid >= _GROUPS)
    def _zero():
        zeros16 = jnp.zeros((16,), jnp.float32)
        for r in range(8):
            def zrow(j, carry):
                vbuf[r, pl.ds(j * 16, 16)] = zeros16
                return carry
            lax.fori_loop(0, CHUNK // 16, zrow, 0)
        for c0, w in _CHUNKS:
            pltpu.sync_copy(vbuf.at[:, pl.ds(0, w)],
                            sample_hbm.at[pl.ds(row, 8), pl.ds(c0, w)])


def _sc_fill(x):
    return pl.kernel(
        _sc_fill_body,
        out_type=[
            jax.ShapeDtypeStruct((ROWS, N), jnp.float32),
            jax.ShapeDtypeStruct((ROWS, N), jnp.float32),
        ],
        mesh=plsc.VectorSubcoreMesh(core_axis_name="c", subcore_axis_name="s"),
        scratch_types=[pltpu.VMEM((8, CHUNK), jnp.float32)],
        compiler_params=pltpu.CompilerParams(use_tc_tiling_on_sc=True),
    )(x)


def _scatter_kernel(idx_v_ref, idx_s_ref, x_any, zeroed_ref, scores0_ref,
                    sample_out, scores_out, obuf, stbuf, tbuf,
                    sem, sem_in, sem_tail):
    cp_in = pltpu.make_async_copy(
        x_any.at[:, pl.ds(NALIGNED, NTAIL)], tbuf, sem_in)
    cp_in.start()

    idx_v = idx_v_ref[...]
    lane = lax.broadcasted_iota(jnp.int32, (ROWS, 128), 1)
    in_main = idx_v < NALIGNED
    obuf[...] = jnp.where((lane == idx_v % 128) & in_main, 1.0, 0.0)

    tail_col = lax.broadcasted_iota(jnp.int32, (ROWS, NTAIL), 1) + NALIGNED
    stbuf[...] = jnp.where(tail_col == idx_v, 1.0, 0.0)
    cp_st = pltpu.make_async_copy(
        stbuf, sample_out.at[:, pl.ds(NALIGNED, NTAIL)], sem_tail)
    cp_st.start()

    def window(r):
        c0 = (idx_s_ref[r, 0] // 128) * 128
        return jnp.minimum(c0, NALIGNED - 128)

    def issue(r, carry):
        pltpu.make_async_copy(
            obuf.at[pl.ds(r, 1), :],
            sample_out.at[pl.ds(r, 1), pl.ds(window(r), 128)],
            sem,
        ).start()
        return carry

    lax.fori_loop(0, ROWS, issue, 0)

    cp_in.wait()
    cp_sc = pltpu.make_async_copy(
        tbuf, scores_out.at[:, pl.ds(NALIGNED, NTAIL)], sem_tail)
    cp_sc.start()

    def drain(r, carry):
        pltpu.make_async_copy(
            obuf.at[pl.ds(r, 1), :],
            sample_out.at[pl.ds(r, 1), pl.ds(window(r), 128)],
            sem,
        ).wait()
        return carry

    lax.fori_loop(0, ROWS, drain, 0)
    cp_st.wait()
    cp_sc.wait()


def _scatter_ones(idx, x, zeroed, scores0):
    return pl.pallas_call(
        _scatter_kernel,
        in_specs=[
            pl.BlockSpec((ROWS, 1), lambda: (0, 0)),
            pl.BlockSpec(memory_space=pltpu.SMEM),
            pl.BlockSpec(memory_space=pl.ANY),
            pl.BlockSpec(memory_space=pl.ANY),
            pl.BlockSpec(memory_space=pl.ANY),
        ],
        out_specs=[
            pl.BlockSpec(memory_space=pl.ANY),
            pl.BlockSpec(memory_space=pl.ANY),
        ],
        out_shape=[
            jax.ShapeDtypeStruct((ROWS, N), jnp.float32),
            jax.ShapeDtypeStruct((ROWS, N), jnp.float32),
        ],
        input_output_aliases={3: 0, 4: 1},
        scratch_shapes=[
            pltpu.VMEM((ROWS, 128), jnp.float32),
            pltpu.VMEM((ROWS, NTAIL), jnp.float32),
            pltpu.VMEM((ROWS, NTAIL), jnp.float32),
            pltpu.SemaphoreType.DMA,
            pltpu.SemaphoreType.DMA,
            pltpu.SemaphoreType.DMA,
        ],
    )(idx, idx, x, zeroed, scores0)


def kernel(x, gumbel_u):
    sample0, scores0 = _sc_fill(x)
    ent, idx = _stats_pass(x, gumbel_u)
    sample, scores = _scatter_ones(idx, x, sample0, scores0)
    return (sample, scores, ent.reshape(ROWS))


# speedup vs baseline: 1.1730x; 1.1550x over previous
import jax
import jax.numpy as jnp
from jax import lax
from jax.experimental import pallas as pl
from jax.experimental.pallas import tpu as pltpu
from jax.experimental.pallas import tpu_sc as plsc

ROWS = 128
N = 100000
NALIGNED = (N // 128) * 128
NTAIL = N - NALIGNED
BR = 16
NBLK = ROWS // BR

_BIG_I32 = 2**30


def _stats_kernel(x_ref, u_ref, ent_ref, idx_ref):
    xb = x_ref[...]
    ub = u_ref[...]

    col = lax.broadcasted_iota(jnp.int32, (BR, N), 1)

    uc = jnp.clip(ub, 1e-10, 1.0 - 1e-10)
    y = xb - jnp.log(-jnp.log(uc))

    lv = jnp.max(y, axis=1, keepdims=True)
    li = jnp.min(jnp.where(y == lv, col, _BIG_I32), axis=1, keepdims=True)
    idx_ref[...] = li

    m = jnp.max(xb, axis=1, keepdims=True)
    e = jnp.exp(xb - m)
    z = jnp.sum(e, axis=1, keepdims=True)
    w = jnp.sum(xb * e, axis=1, keepdims=True)
    ent_ref[...] = m + jnp.log(z) - w / z


def _stats_pass(x, gumbel_u):
    return pl.pallas_call(
        _stats_kernel,
        grid=(NBLK,),
        in_specs=[
            pl.BlockSpec((BR, N), lambda i: (i, 0)),
            pl.BlockSpec((BR, N), lambda i: (i, 0)),
        ],
        out_specs=[
            pl.BlockSpec((BR, 1), lambda i: (i, 0)),
            pl.BlockSpec((BR, 1), lambda i: (i, 0)),
        ],
        out_shape=[
            jax.ShapeDtypeStruct((ROWS, 1), jnp.float32),
            jax.ShapeDtypeStruct((ROWS, 1), jnp.int32),
        ],
    )(x, gumbel_u)


CHUNK = 12800
_CHUNKS = [(c0, min(CHUNK, NALIGNED - c0)) for c0 in range(0, NALIGNED, CHUNK)]
_GROUPS = ROWS // 8


def _sc_fill_body(sample_hbm, vbuf):
    cid = lax.axis_index("c")
    sid = lax.axis_index("s")
    wid = sid * 2 + cid
    row = (wid // 2) * 8
    half = wid % 2

    zeros16 = jnp.zeros((16,), jnp.float32)
    for r in range(8):
        def zrow(j, carry):
            vbuf[r, pl.ds(j * 16, 16)] = zeros16
            return carry
        lax.fori_loop(0, CHUNK // 16, zrow, 0)
    for k, (c0, w) in enumerate(_CHUNKS):
        @pl.when(jnp.int32(k % 2) == half)
        def _put():
            pltpu.sync_copy(vbuf.at[:, pl.ds(0, w)],
                            sample_hbm.at[pl.ds(row, 8), pl.ds(c0, w)])


def _sc_fill():
    return pl.kernel(
        _sc_fill_body,
        out_type=[jax.ShapeDtypeStruct((ROWS, N), jnp.float32)],
        mesh=plsc.VectorSubcoreMesh(core_axis_name="c", subcore_axis_name="s"),
        scratch_types=[pltpu.VMEM((8, CHUNK), jnp.float32)],
        compiler_params=pltpu.CompilerParams(use_tc_tiling_on_sc=True),
    )()


def _scatter_kernel(idx_v_ref, idx_s_ref, zeroed_ref,
                    sample_out, obuf, stbuf, sem, sem_tail):
    idx_v = idx_v_ref[...]
    lane = lax.broadcasted_iota(jnp.int32, (ROWS, 128), 1)
    in_main = idx_v < NALIGNED
    obuf[...] = jnp.where((lane == idx_v % 128) & in_main, 1.0, 0.0)

    tail_col = lax.broadcasted_iota(jnp.int32, (ROWS, NTAIL), 1) + NALIGNED
    stbuf[...] = jnp.where(tail_col == idx_v, 1.0, 0.0)
    cp_st = pltpu.make_async_copy(
        stbuf, sample_out.at[:, pl.ds(NALIGNED, NTAIL)], sem_tail)
    cp_st.start()

    def window(r):
        c0 = (idx_s_ref[r, 0] // 128) * 128
        return jnp.minimum(c0, NALIGNED - 128)

    def issue(r, carry):
        pltpu.make_async_copy(
            obuf.at[pl.ds(r, 1), :],
            sample_out.at[pl.ds(r, 1), pl.ds(window(r), 128)],
            sem,
        ).start()
        return carry

    lax.fori_loop(0, ROWS, issue, 0)

    def drain(r, carry):
        pltpu.make_async_copy(
            obuf.at[pl.ds(r, 1), :],
            sample_out.at[pl.ds(r, 1), pl.ds(window(r), 128)],
            sem,
        ).wait()
        return carry

    lax.fori_loop(0, ROWS, drain, 0)
    cp_st.wait()


def _scatter_ones(idx, zeroed):
    return pl.pallas_call(
        _scatter_kernel,
        in_specs=[
            pl.BlockSpec((ROWS, 1), lambda: (0, 0)),
            pl.BlockSpec(memory_space=pltpu.SMEM),
            pl.BlockSpec(memory_space=pl.ANY),
        ],
        out_specs=pl.BlockSpec(memory_space=pl.ANY),
        out_shape=jax.ShapeDtypeStruct((ROWS, N), jnp.float32),
        input_output_aliases={2: 0},
        scratch_shapes=[
            pltpu.VMEM((ROWS, 128), jnp.float32),
            pltpu.VMEM((ROWS, NTAIL), jnp.float32),
            pltpu.SemaphoreType.DMA,
            pltpu.SemaphoreType.DMA,
        ],
    )(idx, idx, zeroed)


def kernel(x, gumbel_u):
    (sample0,) = _sc_fill()
    ent, idx = _stats_pass(x, gumbel_u)
    sample = _scatter_ones(idx, sample0)
    scores = x * 1.0
    return (sample, scores, ent.reshape(ROWS))
